# Initial kernel scaffold; baseline (speedup 1.0000x reference)
import jax
import jax.numpy as jnp
from jax.experimental import pallas as pl


def kernel(x, conv_extract_w, conv_extract_b, bn1_gamma, bn1_beta, bn1_mean, bn1_var, conv2_w, bn2_gamma, bn2_beta, bn2_mean, bn2_var):
    raise NotImplementedError("write your pallas kernel here")


# speedup vs baseline: 2.3520x; 2.3520x over previous
import jax
import jax.numpy as jnp
from jax.experimental import pallas as pl
from jax.experimental.pallas import tpu as pltpu

B = 2
IN_C = 64
OUT_C = 32
H0 = 96
HU = 2 * H0
NUM = 32
K = 5


S = 24
RS = HU // S
HR = RS // 2


def _dense_body(x_ref, w1_ref, cb_ref, g1_ref, be1_ref, m1_ref, v1_ref,
                w2_ref, g2_ref, be2_ref, m2_ref, v2_ref, out_ref):
    s = pl.program_id(1)
    xb = x_ref[0, :, pl.ds(s * HR, HR + 4), :]
    ctr = xb[:, 1:HR + 3, :]
    upr = xb[:, 0:HR + 2, :]
    dnr = xb[:, 2:HR + 4, :]
    ev2 = 0.75 * ctr + 0.25 * upr
    od2 = 0.75 * ctr + 0.25 * dnr
    xh4 = jnp.stack([ev2, od2], axis=2).reshape(IN_C, RS + 4, H0)
    xh = xh4[:, 1:RS + 3, :]
    left = jnp.concatenate([xh[:, :, :1], xh[:, :, :-1]], axis=2)
    right = jnp.concatenate([xh[:, :, 1:], xh[:, :, -1:]], axis=2)
    ev = 0.75 * xh + 0.25 * left
    od = 0.75 * xh + 0.25 * right
    xuh = jnp.stack([ev, od], axis=3).reshape(IN_C, RS + 2, HU)
    rg = s * RS - 1 + jax.lax.broadcasted_iota(jnp.int32, (1, RS + 2, 1), 1)
    xuh = jnp.where((rg >= 0) & (rg < HU), xuh, 0.0)
    xp = jnp.pad(xuh, ((0, 0), (0, 0), (1, 1))).astype(jnp.bfloat16)
    ps = []
    for k in range(9):
        dy, dx = k // 3, k % 3
        sl = xp[:, dy:dy + RS, dx:dx + HU].reshape(IN_C, RS * HU)
        ps.append(jnp.dot(w1_ref[k], sl, preferred_element_type=jnp.float32))
    while len(ps) > 1:
        nxt = []
        for i in range(0, len(ps) - 1, 2):
            nxt.append(ps[i] + ps[i + 1])
        if len(ps) % 2:
            nxt.append(ps[-1])
        ps = nxt
    xe = ps[0] + cb_ref[...]
    xe = (xe - m1_ref[...]) / jnp.sqrt(v1_ref[...] + 1e-5) * g1_ref[...] + be1_ref[...]
    xe = jnp.maximum(xe, 0.0)
    x2 = jnp.dot(w2_ref[...], xe.astype(jnp.bfloat16),
                 preferred_element_type=jnp.float32)
    x2 = (x2 - m2_ref[...]) / jnp.sqrt(v2_ref[...] + 1e-5) * g2_ref[...] + be2_ref[...]
    xf = jnp.where(x2 > 1.0, x2, 0.0)
    out_ref[0] = xf.reshape(OUT_C, RS, HU)


def _nms_body(p_ref, w_ref, pos_ref):
    p = p_ref[0]
    pp = jnp.pad(p, ((2, 2), (2, 2)))
    cs = pp[:, 0:HU] + pp[:, 1:HU + 1] + pp[:, 2:HU + 2] + pp[:, 3:HU + 3] + pp[:, 4:HU + 4]
    ws = cs[0:HU] + cs[1:HU + 1] + cs[2:HU + 2] + cs[3:HU + 3] + cs[4:HU + 4]
    wsp = jnp.pad(ws, ((2, 2), (2, 2)))
    mc = jnp.maximum(jnp.maximum(jnp.maximum(wsp[:, 0:HU], wsp[:, 1:HU + 1]),
                                 jnp.maximum(wsp[:, 2:HU + 2], wsp[:, 3:HU + 3])),
                     wsp[:, 4:HU + 4])
    mp = jnp.maximum(jnp.maximum(jnp.maximum(mc[0:HU], mc[1:HU + 1]),
                                 jnp.maximum(mc[2:HU + 2], mc[3:HU + 3])),
                     mc[4:HU + 4])
    work = jnp.where(ws == mp, ws, 0.0)
    li = (jax.lax.broadcasted_iota(jnp.int32, (HU, HU), 0) * HU
          + jax.lax.broadcasted_iota(jnp.int32, (HU, HU), 1))
    for i in range(NUM):
        mx = jnp.max(work)
        idx = jnp.min(jnp.where(work == mx, li, jnp.int32(2 ** 30)))
        th = idx // HU
        tw = idx - th * HU
        rsel = (jax.lax.broadcasted_iota(jnp.int32, (K, HU + 4), 1)
                == th + jax.lax.broadcasted_iota(jnp.int32, (K, HU + 4), 0)
                ).astype(jnp.float32)
        csel = (jax.lax.broadcasted_iota(jnp.int32, (HU + 4, K), 0)
                == tw + jax.lax.broadcasted_iota(jnp.int32, (HU + 4, K), 1)
                ).astype(jnp.float32)
        rows = jnp.dot(rsel, pp, preferred_element_type=jnp.float32)
        win = jnp.dot(rows, csel, preferred_element_type=jnp.float32)
        w_ref[0, i] = win
        x1 = jnp.clip(tw - 2, 0, HU - 1)
        y1 = jnp.clip(th - 2, 0, HU - 1)
        x2 = jnp.clip(tw + 2, 0, HU - 1)
        y2 = jnp.clip(th + 2, 0, HU - 1)
        pos_ref[0, i] = jnp.stack([x1, y1, x2, y2]).astype(jnp.int32)
        work = jnp.where(li == idx, jnp.float32(-1.0), work)


def kernel(x, conv_extract_w, conv_extract_b, bn1_gamma, bn1_beta, bn1_mean, bn1_var,
           conv2_w, bn2_gamma, bn2_beta, bn2_mean, bn2_var):
    w1m = jnp.transpose(conv_extract_w, (2, 3, 0, 1)).reshape(9, IN_C, IN_C).astype(jnp.bfloat16)
    w2m = conv2_w[:, :, 0, 0].astype(jnp.bfloat16)

    xpad = jnp.pad(x, ((0, 0), (0, 0), (2, 2), (0, 0)), mode="edge")

    _c64 = pl.BlockSpec((IN_C, 1), lambda b, s: (0, 0))
    _c32 = pl.BlockSpec((OUT_C, 1), lambda b, s: (0, 0))
    xf = pl.pallas_call(
        _dense_body,
        grid=(B, S),
        in_specs=[
            pl.BlockSpec((1, IN_C, H0 + 4, H0), lambda b, s: (b, 0, 0, 0)),
            pl.BlockSpec((9, IN_C, IN_C), lambda b, s: (0, 0, 0)),
            _c64, _c64, _c64, _c64, _c64,
            pl.BlockSpec((OUT_C, IN_C), lambda b, s: (0, 0)),
            _c32, _c32, _c32, _c32,
        ],
        out_specs=pl.BlockSpec((1, OUT_C, RS, HU), lambda b, s: (b, 0, s, 0)),
        out_shape=jax.ShapeDtypeStruct((B, OUT_C, HU, HU), jnp.float32),
    )(xpad, w1m,
      conv_extract_b.reshape(IN_C, 1), bn1_gamma.reshape(IN_C, 1),
      bn1_beta.reshape(IN_C, 1), bn1_mean.reshape(IN_C, 1), bn1_var.reshape(IN_C, 1),
      w2m,
      bn2_gamma.reshape(OUT_C, 1), bn2_beta.reshape(OUT_C, 1),
      bn2_mean.reshape(OUT_C, 1), bn2_var.reshape(OUT_C, 1))

    wins, pos = pl.pallas_call(
        _nms_body,
        grid=(B * OUT_C,),
        in_specs=[pl.BlockSpec((1, HU, HU), lambda i: (i, 0, 0))],
        out_specs=[
            pl.BlockSpec((1, NUM, K, K), lambda i: (i, 0, 0, 0)),
            pl.BlockSpec((1, NUM, 4), lambda i: (i, 0, 0)),
        ],
        out_shape=[
            jax.ShapeDtypeStruct((B * OUT_C, NUM, K, K), jnp.float32),
            jax.ShapeDtypeStruct((B * OUT_C, NUM, 4), jnp.int32),
        ],
    )(xf.reshape(B * OUT_C, HU, HU))

    imp_attns = wins.reshape(B, OUT_C, NUM, K, K)
    imp_loca = pos.reshape(B, OUT_C, NUM, 4)
    return imp_attns, imp_loca
